# baseline (speedup 1.0000x reference)
import functools

import jax
import jax.numpy as jnp
from jax import lax
from jax.experimental import pallas as pl
from jax.experimental.pallas import tpu as pltpu
from jax.experimental.pallas import tpu_sc as plsc

VOCAB = 1000000
EMBED = 64
HIDDEN = 128
BATCH = 16384

NC = 2
NS = 16
NW = NC * NS
B_FLAT = BATCH * 2
B_PER_W = B_FLAT // NW
CHUNK = 128
N_CHUNKS = B_PER_W // CHUNK


def _gather_body(idx_hbm, table_hbm, out_hbm, idx_v, rows_v, sem):
    wid = lax.axis_index("s") * NC + lax.axis_index("c")
    base = wid * B_PER_W
    pltpu.sync_copy(idx_hbm.at[wid], idx_v)
    copies = []
    for j in range(N_CHUNKS):
        copies.append(
            pltpu.async_copy(
                table_hbm.at[idx_v.at[j]],
                rows_v.at[pl.ds(j * CHUNK, CHUNK)],
                sem,
            )
        )
    for c in copies:
        c.wait()
    pltpu.sync_copy(rows_v, out_hbm.at[pl.ds(base, B_PER_W)])


@functools.cache
def _gather_call():
    return pl.kernel(
        _gather_body,
        out_type=jax.ShapeDtypeStruct((B_FLAT, EMBED), jnp.float32),
        mesh=plsc.VectorSubcoreMesh(
            core_axis_name="c", subcore_axis_name="s",
            num_cores=NC, num_subcores=NS,
        ),
        scratch_types=[
            pltpu.VMEM((N_CHUNKS, CHUNK), jnp.int32),
            pltpu.VMEM((B_PER_W, EMBED), jnp.float32),
            pltpu.SemaphoreType.DMA,
        ],
        compiler_params=pltpu.CompilerParams(use_tc_tiling_on_sc=False),
    )


TK = 4096


def _tconv_body(t_ref, eye_ref, o_ref):
    o_ref[...] = lax.dot_general(
        t_ref[...], eye_ref[...], (((0,), (0,)), ((), ())),
        preferred_element_type=jnp.float32,
    )


def _tconv_call(emb_t, eye):
    grid = ((VOCAB + TK - 1) // TK,)
    return pl.pallas_call(
        _tconv_body,
        grid=grid,
        in_specs=[
            pl.BlockSpec((EMBED, TK), lambda i: (0, i)),
            pl.BlockSpec((EMBED, EMBED), lambda i: (0, 0)),
        ],
        out_specs=pl.BlockSpec((TK, EMBED), lambda i: (i, 0)),
        out_shape=jax.ShapeDtypeStruct((VOCAB, EMBED), jnp.float32),
    )(emb_t, eye)


def _mlp_body(c_ref, w1_ref, b1_ref, w2_ref, b2_ref, o_ref):
    c = c_ref[...]
    h = lax.dot_general(
        c, w1_ref[...], (((1,), (1,)), ((), ())),
        preferred_element_type=jnp.float32,
    )
    h = jnp.maximum(h + b1_ref[...], 0.0)
    o = jnp.sum(h * w2_ref[...], axis=1, keepdims=True)
    o_ref[...] = jax.nn.sigmoid(o + b2_ref[0, 0])


BT = 2048


def _mlp_call(combined, W1, b1, W2, b2):
    grid = (BATCH // BT,)
    return pl.pallas_call(
        _mlp_body,
        grid=grid,
        in_specs=[
            pl.BlockSpec((BT, 2 * EMBED), lambda i: (i, 0)),
            pl.BlockSpec((HIDDEN, 2 * EMBED), lambda i: (0, 0)),
            pl.BlockSpec((1, HIDDEN), lambda i: (0, 0)),
            pl.BlockSpec((1, HIDDEN), lambda i: (0, 0)),
            pl.BlockSpec(memory_space=pltpu.SMEM),
        ],
        out_specs=pl.BlockSpec((BT, 1), lambda i: (i, 0)),
        out_shape=jax.ShapeDtypeStruct((BATCH, 1), jnp.float32),
    )(combined, W1, b1, W2, b2)


def kernel(x, emb, W1, b1, W2, b2):
    idx = x.reshape(NW, N_CHUNKS, CHUNK)
    emb_bf = _tconv_call(emb.T, jnp.eye(EMBED, dtype=jnp.float32))
    combined_flat = _gather_call()(idx, emb_bf)
    combined = combined_flat.reshape(BATCH, 2 * EMBED)
    return _mlp_call(
        combined, W1, b1.reshape(1, HIDDEN), W2, b2.reshape(1, 1)
    )


# speedup vs baseline: 2.0126x; 2.0126x over previous
import functools

import jax
import jax.numpy as jnp
from jax import lax
from jax.experimental import pallas as pl
from jax.experimental.pallas import tpu as pltpu
from jax.experimental.pallas import tpu_sc as plsc

VOCAB = 1000000
EMBED = 64
HIDDEN = 128
BATCH = 16384

NC = 2
NS = 16
NW = NC * NS
B_FLAT = BATCH * 2
B_PER_W = B_FLAT // NW
CHUNK = 128
N_CHUNKS = B_PER_W // CHUNK
NBUF = 4


def _gather_body(idx_hbm, table_hbm, out_hbm, idx_v, rows_v, gsem, wsem):
    wid = lax.axis_index("s") * NC + lax.axis_index("c")
    base = wid * B_PER_W
    pltpu.sync_copy(idx_hbm.at[wid], idx_v)
    gc = [None] * N_CHUNKS
    wc = [None] * N_CHUNKS
    for j in range(NBUF):
        gc[j] = pltpu.async_copy(
            table_hbm.at[idx_v.at[j]], rows_v.at[j], gsem
        )
    for j in range(N_CHUNKS):
        gc[j].wait()
        wc[j] = pltpu.async_copy(
            rows_v.at[j % NBUF],
            out_hbm.at[pl.ds(base + j * CHUNK, CHUNK)],
            wsem,
        )
        nj = j + NBUF
        if nj < N_CHUNKS:
            wc[j].wait()
            gc[nj] = pltpu.async_copy(
                table_hbm.at[idx_v.at[nj]], rows_v.at[j % NBUF], gsem
            )
    for j in range(N_CHUNKS - NBUF, N_CHUNKS):
        wc[j].wait()


@functools.cache
def _gather_call():
    return pl.kernel(
        _gather_body,
        out_type=jax.ShapeDtypeStruct((B_FLAT, 2 * EMBED), jnp.float32),
        mesh=plsc.VectorSubcoreMesh(
            core_axis_name="c", subcore_axis_name="s",
            num_cores=NC, num_subcores=NS,
        ),
        scratch_types=[
            pltpu.VMEM((N_CHUNKS, CHUNK), jnp.int32),
            pltpu.VMEM((NBUF, CHUNK, 2 * EMBED), jnp.float32),
            pltpu.SemaphoreType.DMA,
            pltpu.SemaphoreType.DMA,
        ],
    )


TK = 4096
TKH = TK // 2
N_BLOCKS = (VOCAB + TK - 1) // TK
PAIRS = N_BLOCKS * TKH


def _pack_body(t_ref, eye_ref, o_ref):
    a = lax.dot_general(
        t_ref[...], eye_ref[...], (((0,), (0,)), ((), ())),
        preferred_element_type=jnp.float32,
    )
    o_ref[...] = jnp.concatenate([a[:TKH], a[TKH:]], axis=1)


def _pack_call(emb_t, eye):
    return pl.pallas_call(
        _pack_body,
        grid=(N_BLOCKS,),
        in_specs=[
            pl.BlockSpec((EMBED, TK), lambda i: (0, i)),
            pl.BlockSpec((EMBED, EMBED), lambda i: (0, 0)),
        ],
        out_specs=pl.BlockSpec((TKH, 2 * EMBED), lambda i: (i, 0)),
        out_shape=jax.ShapeDtypeStruct((PAIRS, 2 * EMBED), jnp.float32),
    )(emb_t, eye)


def _mlp_body(g0_ref, g1_ref, p_ref, w1_ref, b1_ref, w2_ref, b2_ref, o_ref):
    p = p_ref[...]
    g0 = g0_ref[...]
    g1 = g1_ref[...]
    c0 = jnp.where(p[:, 0:1] == 1, g0[:, EMBED:], g0[:, :EMBED])
    c1 = jnp.where(p[:, 1:2] == 1, g1[:, EMBED:], g1[:, :EMBED])
    c = jnp.concatenate([c0, c1], axis=1)
    h = lax.dot_general(
        c, w1_ref[...], (((1,), (1,)), ((), ())),
        preferred_element_type=jnp.float32,
    )
    h = jnp.maximum(h + b1_ref[...], 0.0)
    o = jnp.sum(h * w2_ref[...], axis=1, keepdims=True)
    o_ref[...] = jax.nn.sigmoid(o + b2_ref[0, 0])


BT = 2048


def _mlp_call(g, par, W1, b1, W2, b2):
    grid = (BATCH // BT,)
    return pl.pallas_call(
        _mlp_body,
        grid=grid,
        in_specs=[
            pl.BlockSpec((BT, 2 * EMBED), lambda i: (i, 0)),
            pl.BlockSpec((BT, 2 * EMBED), lambda i: (i + BATCH // BT, 0)),
            pl.BlockSpec((BT, 2), lambda i: (i, 0)),
            pl.BlockSpec((HIDDEN, 2 * EMBED), lambda i: (0, 0)),
            pl.BlockSpec((1, HIDDEN), lambda i: (0, 0)),
            pl.BlockSpec((1, HIDDEN), lambda i: (0, 0)),
            pl.BlockSpec(memory_space=pltpu.SMEM),
        ],
        out_specs=pl.BlockSpec((BT, 1), lambda i: (i, 0)),
        out_shape=jax.ShapeDtypeStruct((BATCH, 1), jnp.float32),
    )(g, g, par, W1, b1, W2, b2)


def kernel(x, emb, W1, b1, W2, b2):
    par = (x >> 11) & 1
    r = ((x >> 12) << 11) + (x & (TKH - 1))
    idx = jnp.concatenate([r[:, 0], r[:, 1]]).reshape(NW, N_CHUNKS, CHUNK)
    table2 = _pack_call(emb.T, jnp.eye(EMBED, dtype=jnp.float32))
    g = _gather_call()(idx, table2)
    return _mlp_call(
        g, par, W1, b1.reshape(1, HIDDEN), W2, b2.reshape(1, 1)
    )


# speedup vs baseline: 2.6907x; 1.3369x over previous
import functools

import jax
import jax.numpy as jnp
from jax import lax
from jax.experimental import pallas as pl
from jax.experimental.pallas import tpu as pltpu
from jax.experimental.pallas import tpu_sc as plsc

VOCAB = 1000000
EMBED = 64
HIDDEN = 128
BATCH = 16384

NC = 2
NS = 16
NW = NC * NS
B_FLAT = BATCH * 2
B_PER_W = B_FLAT // NW
CHUNK = 128
N_CHUNKS = B_PER_W // CHUNK
NBUF = 4


def _gather_body(idx_hbm, table_hbm, out_hbm, idx_v, rows_v, gsem, wsem):
    wid = lax.axis_index("s") * NC + lax.axis_index("c")
    base = wid * B_PER_W
    pltpu.sync_copy(idx_hbm.at[wid], idx_v)
    gc = [None] * N_CHUNKS
    wc = [None] * N_CHUNKS
    for j in range(NBUF):
        gc[j] = pltpu.async_copy(
            table_hbm.at[idx_v.at[j]], rows_v.at[j], gsem
        )
    for j in range(N_CHUNKS):
        gc[j].wait()
        wc[j] = pltpu.async_copy(
            rows_v.at[j % NBUF],
            out_hbm.at[pl.ds(base + j * CHUNK, CHUNK)],
            wsem,
        )
        nj = j + NBUF
        if nj < N_CHUNKS:
            wc[j].wait()
            gc[nj] = pltpu.async_copy(
                table_hbm.at[idx_v.at[nj]], rows_v.at[j % NBUF], gsem
            )
    for j in range(N_CHUNKS - NBUF, N_CHUNKS):
        wc[j].wait()


@functools.cache
def _gather_call():
    return pl.kernel(
        _gather_body,
        out_type=jax.ShapeDtypeStruct((B_FLAT, 2 * EMBED), jnp.int32),
        mesh=plsc.VectorSubcoreMesh(
            core_axis_name="c", subcore_axis_name="s",
            num_cores=NC, num_subcores=NS,
        ),
        scratch_types=[
            pltpu.VMEM((N_CHUNKS, CHUNK), jnp.int32),
            pltpu.VMEM((NBUF, CHUNK, 2 * EMBED), jnp.int32),
            pltpu.SemaphoreType.DMA,
            pltpu.SemaphoreType.DMA,
        ],
    )


TK = 8192
TKQ = TK // 4
SH = 11
N_BLOCKS = (VOCAB + TK - 1) // TK
QUADS = N_BLOCKS * TKQ


def _rne16(u):
    return (u + 0x7FFF + ((u >> 16) & 1)) >> 16


def _pack_body(t_ref, eye_ref, o_ref):
    a = lax.dot_general(
        t_ref[...], eye_ref[...], (((0,), (0,)), ((), ())),
        preferred_element_type=jnp.float32,
    )
    u = lax.bitcast_convert_type(a, jnp.uint32)
    p01 = _rne16(u[:TKQ]) | (_rne16(u[TKQ:2 * TKQ]) << 16)
    p23 = _rne16(u[2 * TKQ:3 * TKQ]) | (_rne16(u[3 * TKQ:]) << 16)
    o_ref[...] = lax.bitcast_convert_type(
        jnp.concatenate([p01, p23], axis=1), jnp.int32
    )


def _pack_call(emb_t, eye):
    return pl.pallas_call(
        _pack_body,
        grid=(N_BLOCKS,),
        in_specs=[
            pl.BlockSpec((EMBED, TK), lambda i: (0, i)),
            pl.BlockSpec((EMBED, EMBED), lambda i: (0, 0)),
        ],
        out_specs=pl.BlockSpec((TKQ, 2 * EMBED), lambda i: (i, 0)),
        out_shape=jax.ShapeDtypeStruct((QUADS, 2 * EMBED), jnp.int32),
    )(emb_t, eye)


def _unpack(g, s):
    half = jnp.where((s & 2) == 2, g[:, EMBED:], g[:, :EMBED])
    bits = jnp.where((s & 1) == 1, (half >> 16) & 0xFFFF, half & 0xFFFF)
    return lax.bitcast_convert_type(bits << 16, jnp.float32)


def _mlp_body(g0_ref, g1_ref, p_ref, w1_ref, b1_ref, w2_ref, b2_ref, o_ref):
    p = p_ref[...]
    c0 = _unpack(g0_ref[...], p[:, 0:1])
    c1 = _unpack(g1_ref[...], p[:, 1:2])
    c = jnp.concatenate([c0, c1], axis=1)
    h = lax.dot_general(
        c, w1_ref[...], (((1,), (1,)), ((), ())),
        preferred_element_type=jnp.float32,
    )
    h = jnp.maximum(h + b1_ref[...], 0.0)
    o = jnp.sum(h * w2_ref[...], axis=1, keepdims=True)
    o_ref[...] = jax.nn.sigmoid(o + b2_ref[0, 0])


BT = 2048


def _mlp_call(g, par, W1, b1, W2, b2):
    grid = (BATCH // BT,)
    return pl.pallas_call(
        _mlp_body,
        grid=grid,
        in_specs=[
            pl.BlockSpec((BT, 2 * EMBED), lambda i: (i, 0)),
            pl.BlockSpec((BT, 2 * EMBED), lambda i: (i + BATCH // BT, 0)),
            pl.BlockSpec((BT, 2), lambda i: (i, 0)),
            pl.BlockSpec((HIDDEN, 2 * EMBED), lambda i: (0, 0)),
            pl.BlockSpec((1, HIDDEN), lambda i: (0, 0)),
            pl.BlockSpec((1, HIDDEN), lambda i: (0, 0)),
            pl.BlockSpec(memory_space=pltpu.SMEM),
        ],
        out_specs=pl.BlockSpec((BT, 1), lambda i: (i, 0)),
        out_shape=jax.ShapeDtypeStruct((BATCH, 1), jnp.float32),
    )(g, g, par, W1, b1, W2, b2)


def kernel(x, emb, W1, b1, W2, b2):
    par = (x >> SH) & 3
    r = ((x >> (SH + 2)) << SH) + (x & (TKQ - 1))
    idx = jnp.concatenate([r[:, 0], r[:, 1]]).reshape(NW, N_CHUNKS, CHUNK)
    table2 = _pack_call(emb.T, jnp.eye(EMBED, dtype=jnp.float32))
    g = _gather_call()(idx, table2)
    return _mlp_call(
        g, par, W1, b1.reshape(1, HIDDEN), W2, b2.reshape(1, 1)
    )


# speedup vs baseline: 3.0293x; 1.1258x over previous
import functools

import jax
import jax.numpy as jnp
from jax import lax
from jax.experimental import pallas as pl
from jax.experimental.pallas import tpu as pltpu
from jax.experimental.pallas import tpu_sc as plsc

VOCAB = 1000000
EMBED = 64
HIDDEN = 128
BATCH = 16384

NC = 2
NS = 16
NW = NC * NS
B_FLAT = BATCH * 2
B_PER_W = B_FLAT // NW
CHUNK = 128
N_CHUNKS = B_PER_W // CHUNK
NBUF = 6


def _gather_body(idx_hbm, table_hbm, out_hbm, idx_v, rows_v, gsem, wsem):
    wid = lax.axis_index("s") * NC + lax.axis_index("c")
    base = wid * B_PER_W
    pltpu.sync_copy(idx_hbm.at[wid], idx_v)
    gc = [None] * N_CHUNKS
    wc = [None] * N_CHUNKS
    for j in range(NBUF):
        gc[j] = pltpu.async_copy(
            table_hbm.at[idx_v.at[j]], rows_v.at[j], gsem
        )
    for j in range(N_CHUNKS):
        gc[j].wait()
        wc[j] = pltpu.async_copy(
            rows_v.at[j % NBUF],
            out_hbm.at[pl.ds(base + j * CHUNK, CHUNK)],
            wsem,
        )
        nj = j + NBUF
        if nj < N_CHUNKS:
            wc[j].wait()
            gc[nj] = pltpu.async_copy(
                table_hbm.at[idx_v.at[nj]], rows_v.at[j % NBUF], gsem
            )
    for j in range(N_CHUNKS - NBUF, N_CHUNKS):
        wc[j].wait()


@functools.cache
def _gather_call():
    return pl.kernel(
        _gather_body,
        out_type=jax.ShapeDtypeStruct((B_FLAT, 2 * EMBED), jnp.int32),
        mesh=plsc.VectorSubcoreMesh(
            core_axis_name="c", subcore_axis_name="s",
            num_cores=NC, num_subcores=NS,
        ),
        scratch_types=[
            pltpu.VMEM((N_CHUNKS, CHUNK), jnp.int32),
            pltpu.VMEM((NBUF, CHUNK, 2 * EMBED), jnp.int32),
            pltpu.SemaphoreType.DMA,
            pltpu.SemaphoreType.DMA,
        ],
    )


TK = 16384
TKQ = TK // 4
SH = 12
N_BLOCKS = (VOCAB + TK - 1) // TK
QUADS = N_BLOCKS * TKQ


def _rne16(u):
    return (u + 0x7FFF + ((u >> 16) & 1)) >> 16


def _pack_body(t_ref, eye_ref, o_ref):
    a = lax.dot_general(
        t_ref[...], eye_ref[...], (((0,), (0,)), ((), ())),
        preferred_element_type=jnp.float32,
    )
    u = lax.bitcast_convert_type(a, jnp.uint32)
    p01 = _rne16(u[:TKQ]) | (_rne16(u[TKQ:2 * TKQ]) << 16)
    p23 = _rne16(u[2 * TKQ:3 * TKQ]) | (_rne16(u[3 * TKQ:]) << 16)
    o_ref[...] = lax.bitcast_convert_type(
        jnp.concatenate([p01, p23], axis=1), jnp.int32
    )


def _pack_call(emb_t, eye):
    return pl.pallas_call(
        _pack_body,
        grid=(N_BLOCKS,),
        in_specs=[
            pl.BlockSpec((EMBED, TK), lambda i: (0, i)),
            pl.BlockSpec((EMBED, EMBED), lambda i: (0, 0)),
        ],
        out_specs=pl.BlockSpec((TKQ, 2 * EMBED), lambda i: (i, 0)),
        out_shape=jax.ShapeDtypeStruct((QUADS, 2 * EMBED), jnp.int32),
    )(emb_t, eye)


def _unpack(g, s):
    half = jnp.where((s & 2) == 2, g[:, EMBED:], g[:, :EMBED])
    bits = jnp.where((s & 1) == 1, (half >> 16) & 0xFFFF, half & 0xFFFF)
    return lax.bitcast_convert_type(bits << 16, jnp.float32)


def _mlp_body(g0_ref, g1_ref, p_ref, w1_ref, b1_ref, w2_ref, b2_ref, o_ref):
    p = p_ref[...]
    c0 = _unpack(g0_ref[...], p[:, 0:1])
    c1 = _unpack(g1_ref[...], p[:, 1:2])
    c = jnp.concatenate([c0, c1], axis=1)
    h = lax.dot_general(
        c, w1_ref[...], (((1,), (1,)), ((), ())),
        preferred_element_type=jnp.float32,
    )
    h = jnp.maximum(h + b1_ref[...], 0.0)
    o = jnp.sum(h * w2_ref[...], axis=1, keepdims=True)
    o_ref[...] = jax.nn.sigmoid(o + b2_ref[0, 0])


BT = 2048


def _mlp_call(g, par, W1, b1, W2, b2):
    grid = (BATCH // BT,)
    return pl.pallas_call(
        _mlp_body,
        grid=grid,
        in_specs=[
            pl.BlockSpec((BT, 2 * EMBED), lambda i: (i, 0)),
            pl.BlockSpec((BT, 2 * EMBED), lambda i: (i + BATCH // BT, 0)),
            pl.BlockSpec((BT, 2), lambda i: (i, 0)),
            pl.BlockSpec((HIDDEN, 2 * EMBED), lambda i: (0, 0)),
            pl.BlockSpec((1, HIDDEN), lambda i: (0, 0)),
            pl.BlockSpec((1, HIDDEN), lambda i: (0, 0)),
            pl.BlockSpec(memory_space=pltpu.SMEM),
        ],
        out_specs=pl.BlockSpec((BT, 1), lambda i: (i, 0)),
        out_shape=jax.ShapeDtypeStruct((BATCH, 1), jnp.float32),
    )(g, g, par, W1, b1, W2, b2)


def kernel(x, emb, W1, b1, W2, b2):
    par = (x >> SH) & 3
    r = ((x >> (SH + 2)) << SH) + (x & (TKQ - 1))
    idx = jnp.concatenate([r[:, 0], r[:, 1]]).reshape(NW, N_CHUNKS, CHUNK)
    table2 = _pack_call(emb.T, jnp.eye(EMBED, dtype=jnp.float32))
    g = _gather_call()(idx, table2)
    return _mlp_call(
        g, par, W1, b1.reshape(1, HIDDEN), W2, b2.reshape(1, 1)
    )


# speedup vs baseline: 3.1228x; 1.0308x over previous
import functools

import jax
import jax.numpy as jnp
from jax import lax
from jax.experimental import pallas as pl
from jax.experimental.pallas import tpu as pltpu
from jax.experimental.pallas import tpu_sc as plsc

VOCAB = 1000000
EMBED = 64
HIDDEN = 128
BATCH = 16384

NC = 2
NS = 16
NW = NC * NS
B_FLAT = BATCH * 2
B_PER_W = B_FLAT // NW
CHUNK = 128
N_CHUNKS = B_PER_W // CHUNK
NBUF = 7


def _gather_body(idx_hbm, table_hbm, out_hbm, idx_v, rows_v, gsem, wsem):
    wid = lax.axis_index("s") * NC + lax.axis_index("c")
    base = wid * B_PER_W
    pltpu.sync_copy(idx_hbm.at[wid], idx_v)
    gc = [None] * N_CHUNKS
    wc = [None] * N_CHUNKS
    for j in range(NBUF):
        gc[j] = pltpu.async_copy(
            table_hbm.at[idx_v.at[j]], rows_v.at[j], gsem
        )
    for j in range(N_CHUNKS):
        gc[j].wait()
        wc[j] = pltpu.async_copy(
            rows_v.at[j % NBUF],
            out_hbm.at[pl.ds(base + j * CHUNK, CHUNK)],
            wsem,
        )
        nj = j + NBUF
        if nj < N_CHUNKS:
            wc[j].wait()
            gc[nj] = pltpu.async_copy(
                table_hbm.at[idx_v.at[nj]], rows_v.at[j % NBUF], gsem
            )
    for j in range(N_CHUNKS - NBUF, N_CHUNKS):
        wc[j].wait()


@functools.cache
def _gather_call():
    return pl.kernel(
        _gather_body,
        out_type=jax.ShapeDtypeStruct((B_FLAT, 2 * EMBED), jnp.int32),
        mesh=plsc.VectorSubcoreMesh(
            core_axis_name="c", subcore_axis_name="s",
            num_cores=NC, num_subcores=NS,
        ),
        scratch_types=[
            pltpu.VMEM((N_CHUNKS, CHUNK), jnp.int32),
            pltpu.VMEM((NBUF, CHUNK, 2 * EMBED), jnp.int32),
            pltpu.SemaphoreType.DMA,
            pltpu.SemaphoreType.DMA,
        ],
    )


TK = 32768
TKQ = TK // 4
SH = 13
N_BLOCKS = (VOCAB + TK - 1) // TK
QUADS = N_BLOCKS * TKQ


def _rne16(u):
    return (u + 0x7FFF + ((u >> 16) & 1)) >> 16


def _pack_body(t_ref, eye_ref, o_ref):
    a = lax.dot_general(
        t_ref[...], eye_ref[...], (((0,), (0,)), ((), ())),
        preferred_element_type=jnp.float32,
    )
    u = lax.bitcast_convert_type(a, jnp.uint32)
    p01 = _rne16(u[:TKQ]) | (_rne16(u[TKQ:2 * TKQ]) << 16)
    p23 = _rne16(u[2 * TKQ:3 * TKQ]) | (_rne16(u[3 * TKQ:]) << 16)
    o_ref[...] = lax.bitcast_convert_type(
        jnp.concatenate([p01, p23], axis=1), jnp.int32
    )


def _pack_call(emb_t, eye):
    return pl.pallas_call(
        _pack_body,
        grid=(N_BLOCKS,),
        in_specs=[
            pl.BlockSpec((EMBED, TK), lambda i: (0, i)),
            pl.BlockSpec((EMBED, EMBED), lambda i: (0, 0)),
        ],
        out_specs=pl.BlockSpec((TKQ, 2 * EMBED), lambda i: (i, 0)),
        out_shape=jax.ShapeDtypeStruct((QUADS, 2 * EMBED), jnp.int32),
    )(emb_t, eye)


def _unpack(g, s):
    half = jnp.where((s & 2) == 2, g[:, EMBED:], g[:, :EMBED])
    bits = jnp.where((s & 1) == 1, (half >> 16) & 0xFFFF, half & 0xFFFF)
    return lax.bitcast_convert_type(bits << 16, jnp.float32)


def _mlp_body(g0_ref, g1_ref, p_ref, w1_ref, b1_ref, w2_ref, b2_ref, o_ref):
    p = p_ref[...]
    c0 = _unpack(g0_ref[...], p[:, 0:1])
    c1 = _unpack(g1_ref[...], p[:, 1:2])
    c = jnp.concatenate([c0, c1], axis=1)
    h = lax.dot_general(
        c, w1_ref[...], (((1,), (1,)), ((), ())),
        preferred_element_type=jnp.float32,
    )
    h = jnp.maximum(h + b1_ref[...], 0.0)
    o = jnp.sum(h * w2_ref[...], axis=1, keepdims=True)
    o_ref[...] = jax.nn.sigmoid(o + b2_ref[0, 0])


BT = 2048


def _mlp_call(g, par, W1, b1, W2, b2):
    grid = (BATCH // BT,)
    return pl.pallas_call(
        _mlp_body,
        grid=grid,
        in_specs=[
            pl.BlockSpec((BT, 2 * EMBED), lambda i: (i, 0)),
            pl.BlockSpec((BT, 2 * EMBED), lambda i: (i + BATCH // BT, 0)),
            pl.BlockSpec((BT, 2), lambda i: (i, 0)),
            pl.BlockSpec((HIDDEN, 2 * EMBED), lambda i: (0, 0)),
            pl.BlockSpec((1, HIDDEN), lambda i: (0, 0)),
            pl.BlockSpec((1, HIDDEN), lambda i: (0, 0)),
            pl.BlockSpec(memory_space=pltpu.SMEM),
        ],
        out_specs=pl.BlockSpec((BT, 1), lambda i: (i, 0)),
        out_shape=jax.ShapeDtypeStruct((BATCH, 1), jnp.float32),
    )(g, g, par, W1, b1, W2, b2)


def kernel(x, emb, W1, b1, W2, b2):
    par = (x >> SH) & 3
    r = ((x >> (SH + 2)) << SH) + (x & (TKQ - 1))
    idx = jnp.concatenate([r[:, 0], r[:, 1]]).reshape(NW, N_CHUNKS, CHUNK)
    table2 = _pack_call(emb.T, jnp.eye(EMBED, dtype=jnp.float32))
    g = _gather_call()(idx, table2)
    return _mlp_call(
        g, par, W1, b1.reshape(1, HIDDEN), W2, b2.reshape(1, 1)
    )
